# Initial kernel scaffold; baseline (speedup 1.0000x reference)
import jax
import jax.numpy as jnp
from jax.experimental import pallas as pl


def kernel(x, edge_index, batch, ptr, y, W_init, b_init, W0, b0, W1, b1, W2, b2, g0, be0, g1, be1, W_cls, b_cls):
    raise NotImplementedError("write your pallas kernel here")


# speedup vs baseline: 6.7077x; 6.7077x over previous
import functools

import jax
import jax.numpy as jnp
from jax import lax
from jax.experimental import pallas as pl
from jax.experimental.pallas import tpu as pltpu
from jax.experimental.pallas import tpu_sc as plsc

N = 10000
E = 320000
D = 128
LAYERS = 3
NC = 2
NS = 16
NW = NC * NS
EPT = E // NW
K = 80
NCHUNK = EPT // K
NPAD = 10240
RPT = NPAD // NS
ZROWS = 64

_sc_mesh = plsc.VectorSubcoreMesh(core_axis_name="c", subcore_axis_name="s")


@functools.partial(
    pl.kernel,
    out_type=jax.ShapeDtypeStruct((NC, NPAD, D), jnp.float32),
    mesh=_sc_mesh,
    scratch_types=[
        pltpu.VMEM((K,), jnp.int32),
        pltpu.VMEM((K,), jnp.int32),
        pltpu.VMEM((K, D), jnp.float32),
        pltpu.VMEM((ZROWS, D), jnp.float32),
        pltpu.VMEM_SHARED((NPAD, D), jnp.float32),
        pltpu.SemaphoreType.DMA,
    ],
)
def _propagate(hs_hbm, src_hbm, dst_hbm, out_hbm,
               srci, dsti, rows, zbuf, acc, sem):
    c = lax.axis_index("c")
    s = lax.axis_index("s")
    wid = c * NS + s

    zero16 = jnp.zeros((16,), jnp.float32)

    def zbody(t, carry):
        zbuf[t // 8, pl.ds((t % 8) * 16, 16)] = zero16
        return carry

    lax.fori_loop(0, ZROWS * 8, zbody, 0)
    for j in range(RPT // ZROWS):
        pltpu.sync_copy(zbuf, acc.at[pl.ds(s * RPT + j * ZROWS, ZROWS)])

    plsc.subcore_barrier()

    def ebody(t, carry):
        pltpu.sync_copy(src_hbm.at[wid, t], srci)
        pltpu.sync_copy(dst_hbm.at[wid, t], dsti)
        pltpu.async_copy(hs_hbm.at[srci], rows, sem).wait()
        pltpu.sync_copy(rows, acc.at[dsti], add=True)
        return carry

    lax.fori_loop(0, NCHUNK, ebody, 0)

    plsc.subcore_barrier()

    pltpu.sync_copy(acc.at[pl.ds(s * RPT, RPT)],
                    out_hbm.at[c, pl.ds(s * RPT, RPT)])


def _layer_norm(h, g, b):
    mu = jnp.mean(h, axis=-1, keepdims=True)
    var = jnp.var(h, axis=-1, keepdims=True)
    return (h - mu) / jnp.sqrt(var + 1e-5) * g + b


def kernel(x, edge_index, batch, ptr, y, W_init, b_init, W0, b0, W1, b1,
           W2, b2, g0, be0, g1, be1, W_cls, b_cls):
    src = edge_index[0]
    dst = edge_index[1]
    src3 = src.reshape(NW, NCHUNK, K)
    dst3 = dst.reshape(NW, NCHUNK, K)

    deg_in = jax.ops.segment_sum(jnp.ones((E,), jnp.float32), dst,
                                 num_segments=N) + 1.0
    dinv = lax.rsqrt(deg_in)

    deg_out = jax.ops.segment_sum(jnp.ones((E,), jnp.float32), src,
                                  num_segments=N)
    order = jnp.argsort(-deg_out)
    rank = jnp.zeros((N,), jnp.int32).at[order].set(jnp.arange(N, dtype=jnp.int32))
    chunk = N // LAYERS
    chunk_id = (rank >= chunk).astype(jnp.int32) + (rank >= 2 * chunk).astype(jnp.int32)

    h = x @ W_init + b_init
    prev = h
    convs = [(W0, b0), (W1, b1), (W2, b2)]
    lns = [(g0, be0), (g1, be1)]
    for i in range(LAYERS):
        hs = (prev @ convs[i][0]) * dinv[:, None]
        parts = _propagate(hs, src3, dst3)
        agg = parts[0, :N] + parts[1, :N] + hs
        c = dinv[:, None] * agg + convs[i][1]
        if i != LAYERS - 1:
            c = _layer_norm(jax.nn.relu(c), lns[i][0], lns[i][1])
        h = jnp.where((chunk_id == i)[:, None], c, prev)
        prev = h

    pooled = jnp.sum(h, axis=0, keepdims=True) / float(N)
    logits = pooled @ W_cls + b_cls
    return (jax.nn.log_softmax(logits, axis=1), y)


# speedup vs baseline: 9.7717x; 1.4568x over previous
import functools

import jax
import jax.numpy as jnp
from jax import lax
from jax.experimental import pallas as pl
from jax.experimental.pallas import tpu as pltpu
from jax.experimental.pallas import tpu_sc as plsc

N = 10000
E = 320000
D = 128
LAYERS = 3
NC = 2
NS = 16
NW = NC * NS
EPT = E // NW
K = 80
NCHUNK = EPT // K
NPAD = 10240
RPT = NPAD // NS
ZROWS = 64
BLK = 512
GRID = NPAD // BLK

_sc_mesh = plsc.VectorSubcoreMesh(core_axis_name="c", subcore_axis_name="s")


@functools.partial(
    pl.kernel,
    out_type=jax.ShapeDtypeStruct((NC, NPAD, D), jnp.float32),
    mesh=_sc_mesh,
    scratch_types=[
        [pltpu.VMEM((K,), jnp.int32) for _ in range(4)],
        [pltpu.VMEM((K,), jnp.int32) for _ in range(4)],
        [pltpu.VMEM((K, D), jnp.float32) for _ in range(2)],
        pltpu.VMEM((ZROWS, D), jnp.float32),
        pltpu.VMEM_SHARED((NPAD, D), jnp.float32),
        [pltpu.SemaphoreType.DMA for _ in range(4)],
        [pltpu.SemaphoreType.DMA for _ in range(2)],
        [pltpu.SemaphoreType.DMA for _ in range(2)],
    ],
)
def _propagate(hs_hbm, src_hbm, dst_hbm, out_hbm,
               srci, dsti, rows, zbuf, acc, isem, gsem, ssem):
    c = lax.axis_index("c")
    s = lax.axis_index("s")
    wid = c * NS + s

    zero16 = jnp.zeros((16,), jnp.float32)

    def zbody(t, carry):
        zbuf[t // 8, pl.ds((t % 8) * 16, 16)] = zero16
        return carry

    lax.fori_loop(0, ZROWS * 8, zbody, 0)
    for j in range(RPT // ZROWS):
        pltpu.sync_copy(zbuf, acc.at[pl.ds(s * RPT + j * ZROWS, ZROWS)])

    plsc.subcore_barrier()

    def ebase(t):
        return pl.multiple_of(wid * EPT + t * K, 8)

    def issue_idx(t, p):
        pltpu.async_copy(src_hbm.at[pl.ds(ebase(t), K)], srci[p], isem[p])
        pltpu.async_copy(dst_hbm.at[pl.ds(ebase(t), K)], dsti[p], isem[p])

    def chunk(t, b, p, first=False, do_issue=True):
        if not first:
            pltpu.make_async_copy(rows[b], acc.at[dsti[p]],
                                  ssem[b]).wait()
        pltpu.make_async_copy(src_hbm.at[pl.ds(ebase(t), K)], srci[p],
                              isem[p]).wait()
        pltpu.make_async_copy(dst_hbm.at[pl.ds(ebase(t), K)], dsti[p],
                              isem[p]).wait()
        pltpu.async_copy(hs_hbm.at[srci[p]], rows[b], gsem[b]).wait()
        if do_issue:
            issue_idx(t + 2, (p + 2) % 4)
        pltpu.async_copy(rows[b], acc.at[dsti[p]], ssem[b], add=True)

    issue_idx(0, 0)
    issue_idx(1, 1)
    chunk(0, 0, 0, first=True)
    chunk(1, 1, 1, first=True)

    def mbody(u, carry):
        t0 = 2 + 4 * u
        for j in range(4):
            chunk(t0 + j, j % 2, (2 + j) % 4)
        return carry

    lax.fori_loop(0, (NCHUNK - 5) // 4, mbody, 0)
    chunk(NCHUNK - 3, 0, 2)
    chunk(NCHUNK - 2, 1, 3, do_issue=False)
    chunk(NCHUNK - 1, 0, 0, do_issue=False)
    pltpu.make_async_copy(rows[1], acc.at[dsti[(NCHUNK - 2) % 4]],
                          ssem[1]).wait()
    pltpu.make_async_copy(rows[0], acc.at[dsti[(NCHUNK - 1) % 4]],
                          ssem[0]).wait()

    plsc.subcore_barrier()

    pltpu.sync_copy(acc.at[pl.ds(s * RPT, RPT)],
                    out_hbm.at[c, pl.ds(s * RPT, RPT)])


def _row_block(i):
    return (i, 0)


def _const_block(i):
    return (0, 0)


def _prelude_body(x_ref, wi_ref, bi_ref, w0_ref, dd_ref, ds_ref,
                  h0_ref, hs0_ref, dinv_ref, degout_ref):
    i = pl.program_id(0)
    h0 = jnp.dot(x_ref[...], wi_ref[...],
                 preferred_element_type=jnp.float32) + bi_ref[...]
    h0_ref[...] = h0
    deg = dd_ref[0] + dd_ref[1]
    dinv1 = lax.rsqrt(deg[:, :1] + 1.0)
    dinv_ref[...] = jnp.broadcast_to(dinv1, (BLK, 8))
    hs0_ref[...] = jnp.dot(h0, w0_ref[...],
                           preferred_element_type=jnp.float32) * dinv1
    dsum = ds_ref[0] + ds_ref[1]
    row = i * BLK + lax.broadcasted_iota(jnp.int32, (BLK, 1), 0)
    degout_ref[...] = jnp.where(row < N, dsum, 1e9)


def _tc_prelude(x_pad, W_init, b_init, W0, dd, ds):
    return pl.pallas_call(
        _prelude_body,
        grid=(GRID,),
        in_specs=[
            pl.BlockSpec((BLK, D), _row_block),
            pl.BlockSpec((D, D), _const_block),
            pl.BlockSpec((1, D), _const_block),
            pl.BlockSpec((D, D), _const_block),
            pl.BlockSpec((NC, BLK, 8), lambda i: (0, i, 0)),
            pl.BlockSpec((NC, BLK, 8), lambda i: (0, i, 0)),
        ],
        out_specs=[
            pl.BlockSpec((BLK, D), _row_block),
            pl.BlockSpec((BLK, D), _row_block),
            pl.BlockSpec((BLK, 8), _row_block),
            pl.BlockSpec((BLK, 8), _row_block),
        ],
        out_shape=[
            jax.ShapeDtypeStruct((NPAD, D), jnp.float32),
            jax.ShapeDtypeStruct((NPAD, D), jnp.float32),
            jax.ShapeDtypeStruct((NPAD, 8), jnp.float32),
            jax.ShapeDtypeStruct((NPAD, 8), jnp.float32),
        ],
    )(x_pad, W_init, b_init, W0, dd, ds)


def _layer_body(parts_ref, hs_ref, h_ref, dinv_ref, chunk_ref,
                w_ref, b_ref, g_ref, be_ref, hn_ref, hsn_ref, *, layer):
    agg = parts_ref[0] + parts_ref[1] + hs_ref[...]
    dinv1 = dinv_ref[:, :1]
    cval = dinv1 * agg + b_ref[...]
    r = jnp.maximum(cval, 0.0)
    mu = jnp.mean(r, axis=1, keepdims=True)
    dev = r - mu
    var = jnp.mean(dev * dev, axis=1, keepdims=True)
    t = dev * lax.rsqrt(var + 1e-5) * g_ref[...] + be_ref[...]
    hn = jnp.where(chunk_ref[:, :1] == float(layer), t, h_ref[...])
    hn_ref[...] = hn
    hsn_ref[...] = jnp.dot(hn, w_ref[...],
                           preferred_element_type=jnp.float32) * dinv1


def _tc_layer(layer, parts, hs, h, dinv8, chunk8, W_next, b, g, be):
    return pl.pallas_call(
        functools.partial(_layer_body, layer=layer),
        grid=(GRID,),
        in_specs=[
            pl.BlockSpec((NC, BLK, D), lambda i: (0, i, 0)),
            pl.BlockSpec((BLK, D), _row_block),
            pl.BlockSpec((BLK, D), _row_block),
            pl.BlockSpec((BLK, 8), _row_block),
            pl.BlockSpec((BLK, 8), _row_block),
            pl.BlockSpec((D, D), _const_block),
            pl.BlockSpec((1, D), _const_block),
            pl.BlockSpec((1, D), _const_block),
            pl.BlockSpec((1, D), _const_block),
        ],
        out_specs=[
            pl.BlockSpec((BLK, D), _row_block),
            pl.BlockSpec((BLK, D), _row_block),
        ],
        out_shape=[
            jax.ShapeDtypeStruct((NPAD, D), jnp.float32),
            jax.ShapeDtypeStruct((NPAD, D), jnp.float32),
        ],
    )(parts, hs, h, dinv8, chunk8, W_next, b, g, be)


def _final_body(parts_ref, hs_ref, h_ref, dinv_ref, chunk_ref, b_ref,
                pool_ref):
    i = pl.program_id(0)
    agg = parts_ref[0] + parts_ref[1] + hs_ref[...]
    cval = dinv_ref[:, :1] * agg + b_ref[...]
    hn = jnp.where(chunk_ref[:, :1] == 2.0, cval, h_ref[...])
    row = i * BLK + lax.broadcasted_iota(jnp.int32, (BLK, 1), 0)
    hm = jnp.where(row < N, hn, 0.0)
    part = jnp.sum(hm, axis=0, keepdims=True)

    @pl.when(i == 0)
    def _():
        pool_ref[...] = jnp.zeros_like(pool_ref)

    pool_ref[0:1, :] += part


def _tc_final(parts, hs, h, dinv8, chunk8, b):
    return pl.pallas_call(
        _final_body,
        grid=(GRID,),
        in_specs=[
            pl.BlockSpec((NC, BLK, D), lambda i: (0, i, 0)),
            pl.BlockSpec((BLK, D), _row_block),
            pl.BlockSpec((BLK, D), _row_block),
            pl.BlockSpec((BLK, 8), _row_block),
            pl.BlockSpec((BLK, 8), _row_block),
            pl.BlockSpec((1, D), _const_block),
        ],
        out_specs=pl.BlockSpec((8, D), _const_block),
        out_shape=jax.ShapeDtypeStruct((8, D), jnp.float32),
    )(parts, hs, h, dinv8, chunk8, b)


def _cls_body(pool_ref, wc_ref, bc_ref, out_ref):
    pooled = pool_ref[0:1, :] * (1.0 / N)
    logits = jnp.dot(pooled, wc_ref[...],
                     preferred_element_type=jnp.float32) + bc_ref[...]
    m = jnp.max(logits, axis=1, keepdims=True)
    z = logits - m
    lse = jnp.log(jnp.sum(jnp.exp(z), axis=1, keepdims=True)) + m
    out_ref[...] = logits - lse


def _tc_cls(pool, W_cls, b_cls):
    return pl.pallas_call(
        _cls_body,
        out_shape=jax.ShapeDtypeStruct((1, D), jnp.float32),
    )(pool, W_cls, b_cls)


def kernel(x, edge_index, batch, ptr, y, W_init, b_init, W0, b0, W1, b1,
           W2, b2, g0, be0, g1, be1, W_cls, b_cls):
    src = edge_index[0]
    dst = edge_index[1]
    x_pad = jnp.pad(x, ((0, NPAD - N), (0, 0)))

    deg_in = jax.ops.segment_sum(jnp.ones((E,), jnp.float32), dst,
                                 num_segments=N)
    deg_out = jax.ops.segment_sum(jnp.ones((E,), jnp.float32), src,
                                  num_segments=N)
    order = jnp.argsort(-deg_out)
    rank = jnp.zeros((N,), jnp.int32).at[order].set(
        jnp.arange(N, dtype=jnp.int32))
    chunk = N // LAYERS
    chunk_id = ((rank >= chunk).astype(jnp.float32)
                + (rank >= 2 * chunk).astype(jnp.float32))
    chunk8 = jnp.pad(jnp.broadcast_to(chunk_id[:, None], (N, 8)),
                     ((0, NPAD - N), (0, 0)))
    dd = jnp.stack([jnp.pad(deg_in, (0, NPAD - N))[:, None] *
                    jnp.ones((1, 8), jnp.float32),
                    jnp.zeros((NPAD, 8), jnp.float32)])
    ds = jnp.stack([jnp.pad(deg_out, (0, NPAD - N))[:, None] *
                    jnp.ones((1, 8), jnp.float32),
                    jnp.zeros((NPAD, 8), jnp.float32)])

    bi = b_init.reshape(1, D)
    h, hs, dinv8, _degout8 = _tc_prelude(x_pad, W_init, bi, W0, dd, ds)

    lns = [(g0, be0), (g1, be1)]
    nxt = [W1, W2]
    bs = [b0, b1, b2]
    for i in range(2):
        parts = _propagate(hs, src, dst)
        h, hs = _tc_layer(i, parts, hs, h, dinv8, chunk8,
                          nxt[i], bs[i].reshape(1, D),
                          lns[i][0].reshape(1, D), lns[i][1].reshape(1, D))
    parts = _propagate(hs, src, dst)
    pool = _tc_final(parts, hs, h, dinv8, chunk8, b2.reshape(1, D))
    out = _tc_cls(pool, W_cls, b_cls.reshape(1, D))
    return (out, y)


# speedup vs baseline: 17.8111x; 1.8227x over previous
import functools

import jax
import jax.numpy as jnp
from jax import lax
from jax.experimental import pallas as pl
from jax.experimental.pallas import tpu as pltpu
from jax.experimental.pallas import tpu_sc as plsc

N = 10000
E = 320000
D = 128
LAYERS = 3
NC = 2
NS = 16
NW = NC * NS
EPT = E // NW
K = 80
NCHUNK = EPT // K
NPAD = 10240
RPT = NPAD // NS
ZROWS = 64
BLK = 512
GRID = NPAD // BLK

_sc_mesh = plsc.VectorSubcoreMesh(core_axis_name="c", subcore_axis_name="s")


@functools.partial(
    pl.kernel,
    out_type=jax.ShapeDtypeStruct((NC, NPAD, D), jnp.float32),
    mesh=_sc_mesh,
    scratch_types=[
        [pltpu.VMEM((K,), jnp.int32) for _ in range(4)],
        [pltpu.VMEM((K,), jnp.int32) for _ in range(4)],
        [pltpu.VMEM((K, D), jnp.float32) for _ in range(2)],
        pltpu.VMEM((ZROWS, D), jnp.float32),
        pltpu.VMEM_SHARED((NPAD, D), jnp.float32),
        [pltpu.SemaphoreType.DMA for _ in range(4)],
        [pltpu.SemaphoreType.DMA for _ in range(2)],
        [pltpu.SemaphoreType.DMA for _ in range(2)],
    ],
)
def _propagate(hs_hbm, src_hbm, dst_hbm, out_hbm,
               srci, dsti, rows, zbuf, acc, isem, gsem, ssem):
    c = lax.axis_index("c")
    s = lax.axis_index("s")
    wid = c * NS + s

    zero16 = jnp.zeros((16,), jnp.float32)

    def zbody(t, carry):
        zbuf[t // 8, pl.ds((t % 8) * 16, 16)] = zero16
        return carry

    lax.fori_loop(0, ZROWS * 8, zbody, 0)
    for j in range(RPT // ZROWS):
        pltpu.sync_copy(zbuf, acc.at[pl.ds(s * RPT + j * ZROWS, ZROWS)])

    plsc.subcore_barrier()

    def ebase(t):
        return pl.multiple_of(wid * EPT + t * K, 8)

    def issue_idx(t, p):
        pltpu.async_copy(src_hbm.at[pl.ds(ebase(t), K)], srci[p], isem[p])
        pltpu.async_copy(dst_hbm.at[pl.ds(ebase(t), K)], dsti[p], isem[p])

    def chunk(t, b, p, first=False, do_issue=True):
        if not first:
            pltpu.make_async_copy(rows[b], acc.at[dsti[p]],
                                  ssem[b]).wait()
        pltpu.make_async_copy(src_hbm.at[pl.ds(ebase(t), K)], srci[p],
                              isem[p]).wait()
        pltpu.make_async_copy(dst_hbm.at[pl.ds(ebase(t), K)], dsti[p],
                              isem[p]).wait()
        pltpu.async_copy(hs_hbm.at[srci[p]], rows[b], gsem[b]).wait()
        if do_issue:
            issue_idx(t + 2, (p + 2) % 4)
        pltpu.async_copy(rows[b], acc.at[dsti[p]], ssem[b], add=True)

    issue_idx(0, 0)
    issue_idx(1, 1)
    chunk(0, 0, 0, first=True)
    chunk(1, 1, 1, first=True)

    def mbody(u, carry):
        t0 = 2 + 4 * u
        for j in range(4):
            chunk(t0 + j, j % 2, (2 + j) % 4)
        return carry

    lax.fori_loop(0, (NCHUNK - 5) // 4, mbody, 0)
    chunk(NCHUNK - 3, 0, 2)
    chunk(NCHUNK - 2, 1, 3, do_issue=False)
    chunk(NCHUNK - 1, 0, 0, do_issue=False)
    pltpu.make_async_copy(rows[1], acc.at[dsti[(NCHUNK - 2) % 4]],
                          ssem[1]).wait()
    pltpu.make_async_copy(rows[0], acc.at[dsti[(NCHUNK - 1) % 4]],
                          ssem[0]).wait()

    plsc.subcore_barrier()

    pltpu.sync_copy(acc.at[pl.ds(s * RPT, RPT)],
                    out_hbm.at[c, pl.ds(s * RPT, RPT)])


@functools.partial(
    pl.kernel,
    out_type=tuple(jax.ShapeDtypeStruct((NPAD * 8,), jnp.float32)
                   for _ in range(4)),
    mesh=_sc_mesh,
    scratch_types=[
        [pltpu.VMEM((K,), jnp.int32) for _ in range(4)],
        [pltpu.VMEM((K,), jnp.int32) for _ in range(4)],
        pltpu.VMEM((K,), jnp.float32),
        pltpu.VMEM((RPT,), jnp.float32),
        pltpu.VMEM((RPT * 8,), jnp.float32),
        pltpu.VMEM_SHARED((NPAD,), jnp.float32),
        pltpu.VMEM_SHARED((NPAD,), jnp.float32),
        [pltpu.SemaphoreType.DMA for _ in range(4)],
        [pltpu.SemaphoreType.DMA for _ in range(4)],
        [pltpu.SemaphoreType.DMA for _ in range(4)],
    ],
)
def _degrees(src_hbm, dst_hbm, outd0_hbm, outd1_hbm, outs0_hbm, outs1_hbm,
             ibd, ibs, ones, rtmp, rep, accd, accs, isem, dsem, ssem):
    c = lax.axis_index("c")
    s = lax.axis_index("s")
    wid = c * NS + s
    iota16 = lax.iota(jnp.int32, 16)
    zero16 = jnp.zeros((16,), jnp.float32)
    ones16 = jnp.ones((16,), jnp.float32)

    def ob(t, carry):
        ones[pl.ds(t * 16, 16)] = ones16
        return carry

    lax.fori_loop(0, K // 16, ob, 0)

    def zb(t, carry):
        rtmp[pl.ds(t * 16, 16)] = zero16
        return carry

    lax.fori_loop(0, RPT // 16, zb, 0)
    pltpu.sync_copy(rtmp, accd.at[pl.ds(s * RPT, RPT)])
    pltpu.sync_copy(rtmp, accs.at[pl.ds(s * RPT, RPT)])
    plsc.subcore_barrier()

    def ebase(t):
        return pl.multiple_of(wid * EPT + t * K, 8)

    def issue_idx(t, p):
        pltpu.async_copy(dst_hbm.at[pl.ds(ebase(t), K)], ibd[p], isem[p])
        pltpu.async_copy(src_hbm.at[pl.ds(ebase(t), K)], ibs[p], isem[p])

    def chunk(t, p, do_wait=True, do_issue=True):
        pltpu.make_async_copy(dst_hbm.at[pl.ds(ebase(t), K)], ibd[p],
                              isem[p]).wait()
        pltpu.make_async_copy(src_hbm.at[pl.ds(ebase(t), K)], ibs[p],
                              isem[p]).wait()
        pltpu.async_copy(ones, accd.at[ibd[p]], dsem[p], add=True)
        pltpu.async_copy(ones, accs.at[ibs[p]], ssem[p], add=True)
        p2 = (p + 2) % 4
        if do_wait:
            pltpu.make_async_copy(ones, accd.at[ibd[p2]], dsem[p2]).wait()
            pltpu.make_async_copy(ones, accs.at[ibs[p2]], ssem[p2]).wait()
        if do_issue:
            issue_idx(t + 2, p2)

    issue_idx(0, 0)
    issue_idx(1, 1)
    chunk(0, 0, do_wait=False)
    chunk(1, 1, do_wait=False)

    def mbody(u, carry):
        t0 = 2 + 4 * u
        for j in range(4):
            chunk(t0 + j, (2 + j) % 4)
        return carry

    lax.fori_loop(0, (NCHUNK - 5) // 4, mbody, 0)
    chunk(NCHUNK - 3, 2)
    chunk(NCHUNK - 2, 3, do_issue=False)
    chunk(NCHUNK - 1, 0, do_issue=False)
    for p in (3, 0):
        pltpu.make_async_copy(ones, accd.at[ibd[p]], dsem[p]).wait()
        pltpu.make_async_copy(ones, accs.at[ibs[p]], ssem[p]).wait()

    plsc.subcore_barrier()

    def rw(acc, out0_hbm, out1_hbm):
        pltpu.sync_copy(acc.at[pl.ds(s * RPT, RPT)], rtmp)

        def rb(q, carry):
            v = rtmp[pl.ds(q * 16, 16)]
            for r in range(8):
                rv = jnp.where(iota16 < 8, v[2 * r], v[2 * r + 1])
                rep[pl.ds(q * 128 + r * 16, 16)] = rv
            return carry

        lax.fori_loop(0, RPT // 16, rb, 0)
        stripe = pl.ds(s * RPT * 8, RPT * 8)

        @pl.when(c == 0)
        def _w0():
            pltpu.sync_copy(rep, out0_hbm.at[stripe])

        @pl.when(c == 1)
        def _w1():
            pltpu.sync_copy(rep, out1_hbm.at[stripe])

    rw(accd, outd0_hbm, outd1_hbm)
    rw(accs, outs0_hbm, outs1_hbm)


def _vsum(v):
    t = v[0]
    for j in range(1, 16):
        t = t + v[j]
    return t


def _prefix(v, iota16):
    acc = v[0]
    out = jnp.where(iota16 == 0, acc, 0.0)
    for j in range(1, 16):
        acc = acc + v[j]
        out = out + jnp.where(iota16 == j, acc, 0.0)
    return out


CAP = 8192
HTOT = 8448
CHK = N // LAYERS
THR0 = float(N - CHK)
THR1 = float(N - 2 * CHK)


@functools.partial(
    pl.kernel,
    out_type=jax.ShapeDtypeStruct((NPAD * 8,), jnp.float32),
    mesh=_sc_mesh,
    scratch_types=[
        pltpu.VMEM((RPT * 8,), jnp.float32),
        pltpu.VMEM((RPT,), jnp.float32),
        pltpu.VMEM((CAP,), jnp.float32),
        pltpu.VMEM((CAP,), jnp.float32),
        pltpu.VMEM((RPT * 8,), jnp.float32),
        pltpu.VMEM((16,), jnp.float32),
        pltpu.VMEM((16,), jnp.float32),
        pltpu.VMEM((NS * 16,), jnp.float32),
        pltpu.VMEM((16,), jnp.int32),
        pltpu.VMEM_SHARED((HTOT,), jnp.float32),
        pltpu.VMEM_SHARED((16,), jnp.float32),
        pltpu.VMEM_SHARED((NS * 16,), jnp.float32),
    ],
)
def _centrality(deg_hbm, out_hbm,
                degbuf, degv, histbuf, sbuf, outbuf, tmp16, ones1,
                cntbuf, ivbuf, hist, bcast, counts):
    c = lax.axis_index("c")
    s = lax.axis_index("s")
    zero16 = jnp.zeros((16,), jnp.float32)
    iota16 = lax.iota(jnp.int32, 16)

    @pl.when(c == 0)
    def _stage():
        tmp16[...] = zero16

        def zh(t, carry):
            pltpu.sync_copy(tmp16, hist.at[pl.ds(s * (HTOT // NS) + t * 16,
                                                 16)])
            return carry

        lax.fori_loop(0, HTOT // NS // 16, zh, 0)
        pltpu.sync_copy(deg_hbm.at[pl.ds(s * RPT * 8, RPT * 8)], degbuf)
        ones1[...] = jnp.ones((16,), jnp.float32)

        def ex(q, carry):
            v = jnp.zeros((16,), jnp.float32)
            for j in range(8):
                w = degbuf[pl.ds(q * 128 + j * 16, 16)]
                v = v + jnp.where(iota16 == 2 * j, w[0], 0.0)
                v = v + jnp.where(iota16 == 2 * j + 1, w[8], 0.0)
            degv[pl.ds(q * 16, 16)] = v
            return carry

        lax.fori_loop(0, RPT // 16, ex, 0)

    plsc.subcore_barrier()

    @pl.when(c == 0)
    def _histo():
        def hb(q, carry):
            v = degv[pl.ds(q * 16, 16)]
            ivbuf[...] = jnp.minimum(v.astype(jnp.int32), CAP)
            pltpu.sync_copy(ones1, hist.at[ivbuf], add=True)
            return carry

        lax.fori_loop(0, RPT // 16, hb, 0)

    plsc.subcore_barrier()

    @pl.when(jnp.logical_and(c == 0, s == 0))
    def _scan():
        pltpu.sync_copy(hist.at[pl.ds(0, CAP)], histbuf)

        def sb(q, carry):
            sacc, t0a, t1a = carry
            hv = histbuf[pl.ds(q * 16, 16)]
            sl = _prefix(hv, iota16) + sacc
            sbuf[pl.ds(q * 16, 16)] = sl
            t0a = t0a + jnp.where(sl <= THR0, 1.0, 0.0)
            t1a = t1a + jnp.where(sl <= THR1, 1.0, 0.0)
            return (sl[15], t0a, t1a)

        sacc, t0a, t1a = lax.fori_loop(0, CAP // 16, sb,
                                       (0.0, zero16, zero16))
        t0 = _vsum(t0a)
        t1 = _vsum(t1a)
        def _sread(tf):
            ti = tf.astype(jnp.int32)
            base = (ti // 16) * 16
            lane = ti - base
            v = sbuf[pl.ds(base, 16)]
            val = 0.0
            for j in range(16):
                val = jnp.where(lane == j, v[j], val)
            return val

        cgt0 = float(N) - _sread(t0)
        cgt1 = float(N) - _sread(t1)
        vec = jnp.where(iota16 == 0, t0,
                        jnp.where(iota16 == 1, t1,
                                  jnp.where(iota16 == 2, cgt0, cgt1)))
        tmp16[...] = vec
        pltpu.sync_copy(tmp16, bcast)

    plsc.subcore_barrier()

    @pl.when(c == 0)
    def _counts():
        pltpu.sync_copy(bcast, tmp16)
        tv = tmp16[...]
        t0f = tv[0]
        t1f = tv[1]

        def cb(q, carry):
            e0a, e1a = carry
            v = degv[pl.ds(q * 16, 16)]
            e0a = e0a + jnp.where(v == t0f, 1.0, 0.0)
            e1a = e1a + jnp.where(v == t1f, 1.0, 0.0)
            return (e0a, e1a)

        e0a, e1a = lax.fori_loop(0, RPT // 16, cb, (zero16, zero16))
        e0 = _vsum(e0a)
        e1 = _vsum(e1a)
        vec = jnp.where(iota16 == 0, e0, jnp.where(iota16 == 1, e1, 0.0))
        tmp16[...] = vec
        pltpu.sync_copy(tmp16, counts.at[pl.ds(s * 16, 16)])

    plsc.subcore_barrier()

    @pl.when(c == 0)
    def _assign():
        pltpu.sync_copy(counts, cntbuf)
        pltpu.sync_copy(bcast, tmp16)
        tv = tmp16[...]
        t0f = tv[0]
        t1f = tv[1]
        cgt0 = tv[2]
        cgt1 = tv[3]
        c0 = 0.0
        c1 = 0.0
        for t in range(NS):
            inrange = jnp.where(t < s, 1.0, 0.0)
            cv = cntbuf[pl.ds(t * 16, 16)]
            c0 = c0 + inrange * cv[0]
            c1 = c1 + inrange * cv[1]

        def fb(q, carry):
            c0, c1 = carry
            v = degv[pl.ds(q * 16, 16)]
            e0 = jnp.where(v == t0f, 1.0, 0.0)
            e1 = jnp.where(v == t1f, 1.0, 0.0)
            i0 = _prefix(e0, iota16)
            i1 = _prefix(e1, iota16)
            pos0 = c0 + i0 - e0
            pos1 = c1 + i1 - e1
            rank = jnp.where(e0 == 1.0, cgt0 + pos0,
                             jnp.where(e1 == 1.0, cgt1 + pos1,
                                       jnp.where(v > t0f, 0.0,
                                                 jnp.where(v < t1f,
                                                           9999.0, 4000.0))))
            ch = (jnp.where(rank >= float(CHK), 1.0, 0.0)
                  + jnp.where(rank >= float(2 * CHK), 1.0, 0.0))
            for r in range(8):
                outbuf[pl.ds(q * 128 + r * 16, 16)] = jnp.where(
                    iota16 < 8, ch[2 * r], ch[2 * r + 1])
            return (c0 + i0[15], c1 + i1[15])

        lax.fori_loop(0, RPT // 16, fb, (c0, c1))
        pltpu.sync_copy(outbuf, out_hbm.at[pl.ds(s * RPT * 8, RPT * 8)])


def _row_block(i):
    return (i, 0)


def _const_block(i):
    return (0, 0)


def _prelude_body(x_ref, wi_ref, bi_ref, w0_ref, dd0_ref, dd1_ref,
                  ds0_ref, ds1_ref,
                  h0_ref, hs0_ref, dinv_ref, degout_ref):
    i = pl.program_id(0)
    h0 = jnp.dot(x_ref[...], wi_ref[...],
                 preferred_element_type=jnp.float32) + bi_ref[...]
    h0_ref[...] = h0
    deg = dd0_ref[...] + dd1_ref[...]
    dinv1 = lax.rsqrt(deg[:, :1] + 1.0)
    dinv_ref[...] = jnp.broadcast_to(dinv1, (BLK, 8))
    hs0_ref[...] = jnp.dot(h0, w0_ref[...],
                           preferred_element_type=jnp.float32) * dinv1
    dsum = ds0_ref[...] + ds1_ref[...]
    row = i * BLK + lax.broadcasted_iota(jnp.int32, (BLK, 1), 0)
    degout_ref[...] = jnp.where(row < N, dsum, 1e9)


def _tc_prelude(x_pad, W_init, b_init, W0, dd0, dd1, ds0, ds1):
    return pl.pallas_call(
        _prelude_body,
        grid=(GRID,),
        in_specs=[
            pl.BlockSpec((BLK, D), _row_block),
            pl.BlockSpec((D, D), _const_block),
            pl.BlockSpec((1, D), _const_block),
            pl.BlockSpec((D, D), _const_block),
            pl.BlockSpec((BLK, 8), _row_block),
            pl.BlockSpec((BLK, 8), _row_block),
            pl.BlockSpec((BLK, 8), _row_block),
            pl.BlockSpec((BLK, 8), _row_block),
        ],
        out_specs=[
            pl.BlockSpec((BLK, D), _row_block),
            pl.BlockSpec((BLK, D), _row_block),
            pl.BlockSpec((BLK, 8), _row_block),
            pl.BlockSpec((BLK, 8), _row_block),
        ],
        out_shape=[
            jax.ShapeDtypeStruct((NPAD, D), jnp.float32),
            jax.ShapeDtypeStruct((NPAD, D), jnp.float32),
            jax.ShapeDtypeStruct((NPAD, 8), jnp.float32),
            jax.ShapeDtypeStruct((NPAD, 8), jnp.float32),
        ],
    )(x_pad, W_init, b_init, W0, dd0, dd1, ds0, ds1)


def _layer_body(parts_ref, hs_ref, h_ref, dinv_ref, chunk_ref,
                w_ref, b_ref, g_ref, be_ref, hn_ref, hsn_ref, *, layer):
    agg = parts_ref[0] + parts_ref[1] + hs_ref[...]
    dinv1 = dinv_ref[:, :1]
    cval = dinv1 * agg + b_ref[...]
    r = jnp.maximum(cval, 0.0)
    mu = jnp.mean(r, axis=1, keepdims=True)
    dev = r - mu
    var = jnp.mean(dev * dev, axis=1, keepdims=True)
    t = dev * lax.rsqrt(var + 1e-5) * g_ref[...] + be_ref[...]
    hn = jnp.where(chunk_ref[:, :1] == float(layer), t, h_ref[...])
    hn_ref[...] = hn
    hsn_ref[...] = jnp.dot(hn, w_ref[...],
                           preferred_element_type=jnp.float32) * dinv1


def _tc_layer(layer, parts, hs, h, dinv8, chunk8, W_next, b, g, be):
    return pl.pallas_call(
        functools.partial(_layer_body, layer=layer),
        grid=(GRID,),
        in_specs=[
            pl.BlockSpec((NC, BLK, D), lambda i: (0, i, 0)),
            pl.BlockSpec((BLK, D), _row_block),
            pl.BlockSpec((BLK, D), _row_block),
            pl.BlockSpec((BLK, 8), _row_block),
            pl.BlockSpec((BLK, 8), _row_block),
            pl.BlockSpec((D, D), _const_block),
            pl.BlockSpec((1, D), _const_block),
            pl.BlockSpec((1, D), _const_block),
            pl.BlockSpec((1, D), _const_block),
        ],
        out_specs=[
            pl.BlockSpec((BLK, D), _row_block),
            pl.BlockSpec((BLK, D), _row_block),
        ],
        out_shape=[
            jax.ShapeDtypeStruct((NPAD, D), jnp.float32),
            jax.ShapeDtypeStruct((NPAD, D), jnp.float32),
        ],
    )(parts, hs, h, dinv8, chunk8, W_next, b, g, be)


def _final_body(parts_ref, hs_ref, h_ref, dinv_ref, chunk_ref, b_ref,
                pool_ref):
    i = pl.program_id(0)
    agg = parts_ref[0] + parts_ref[1] + hs_ref[...]
    cval = dinv_ref[:, :1] * agg + b_ref[...]
    hn = jnp.where(chunk_ref[:, :1] == 2.0, cval, h_ref[...])
    row = i * BLK + lax.broadcasted_iota(jnp.int32, (BLK, 1), 0)
    hm = jnp.where(row < N, hn, 0.0)
    part = jnp.sum(hm, axis=0, keepdims=True)

    @pl.when(i == 0)
    def _():
        pool_ref[...] = jnp.zeros_like(pool_ref)

    pool_ref[0:1, :] += part


def _tc_final(parts, hs, h, dinv8, chunk8, b):
    return pl.pallas_call(
        _final_body,
        grid=(GRID,),
        in_specs=[
            pl.BlockSpec((NC, BLK, D), lambda i: (0, i, 0)),
            pl.BlockSpec((BLK, D), _row_block),
            pl.BlockSpec((BLK, D), _row_block),
            pl.BlockSpec((BLK, 8), _row_block),
            pl.BlockSpec((BLK, 8), _row_block),
            pl.BlockSpec((1, D), _const_block),
        ],
        out_specs=pl.BlockSpec((8, D), _const_block),
        out_shape=jax.ShapeDtypeStruct((8, D), jnp.float32),
    )(parts, hs, h, dinv8, chunk8, b)


def _cls_body(pool_ref, wc_ref, bc_ref, out_ref):
    pooled = pool_ref[0:1, :] * (1.0 / N)
    logits = jnp.dot(pooled, wc_ref[...],
                     preferred_element_type=jnp.float32) + bc_ref[...]
    m = jnp.max(logits, axis=1, keepdims=True)
    z = logits - m
    lse = jnp.log(jnp.sum(jnp.exp(z), axis=1, keepdims=True)) + m
    out_ref[...] = logits - lse


def _tc_cls(pool, W_cls, b_cls):
    return pl.pallas_call(
        _cls_body,
        out_shape=jax.ShapeDtypeStruct((1, D), jnp.float32),
    )(pool, W_cls, b_cls)


def kernel(x, edge_index, batch, ptr, y, W_init, b_init, W0, b0, W1, b1,
           W2, b2, g0, be0, g1, be1, W_cls, b_cls):
    src = edge_index[0]
    dst = edge_index[1]
    x_pad = jnp.pad(x, ((0, NPAD - N), (0, 0)))

    dd0, dd1, ds0, ds1 = _degrees(src, dst)
    dd0 = dd0.reshape(NPAD, 8)
    dd1 = dd1.reshape(NPAD, 8)
    ds0 = ds0.reshape(NPAD, 8)
    ds1 = ds1.reshape(NPAD, 8)

    bi = b_init.reshape(1, D)
    h, hs, dinv8, degout8 = _tc_prelude(x_pad, W_init, bi, W0,
                                        dd0, dd1, ds0, ds1)
    chunk8 = _centrality(degout8.reshape(NPAD * 8)).reshape(NPAD, 8)

    lns = [(g0, be0), (g1, be1)]
    nxt = [W1, W2]
    bs = [b0, b1, b2]
    for i in range(2):
        parts = _propagate(hs, src, dst)
        h, hs = _tc_layer(i, parts, hs, h, dinv8, chunk8,
                          nxt[i], bs[i].reshape(1, D),
                          lns[i][0].reshape(1, D), lns[i][1].reshape(1, D))
    parts = _propagate(hs, src, dst)
    pool = _tc_final(parts, hs, h, dinv8, chunk8, b2.reshape(1, D))
    out = _tc_cls(pool, W_cls, b_cls.reshape(1, D))
    return (out, y)


# speedup vs baseline: 17.8490x; 1.0021x over previous
import functools

import jax
import jax.numpy as jnp
from jax import lax
from jax.experimental import pallas as pl
from jax.experimental.pallas import tpu as pltpu
from jax.experimental.pallas import tpu_sc as plsc

N = 10000
E = 320000
D = 128
LAYERS = 3
NC = 2
NS = 16
NW = NC * NS
EPT = E // NW
K = 80
NCHUNK = EPT // K
NPAD = 10240
RPT = NPAD // NS
ZROWS = 64
BLK = 512
GRID = NPAD // BLK

_sc_mesh = plsc.VectorSubcoreMesh(core_axis_name="c", subcore_axis_name="s")


@functools.partial(
    pl.kernel,
    out_type=jax.ShapeDtypeStruct((NC, NPAD, D), jnp.float32),
    mesh=_sc_mesh,
    scratch_types=[
        [pltpu.VMEM((K,), jnp.int32) for _ in range(4)],
        [pltpu.VMEM((K,), jnp.int32) for _ in range(4)],
        [pltpu.VMEM((K, D), jnp.float32) for _ in range(2)],
        pltpu.VMEM((ZROWS, D), jnp.float32),
        pltpu.VMEM_SHARED((NPAD, D), jnp.float32),
        [pltpu.SemaphoreType.DMA for _ in range(4)],
        [pltpu.SemaphoreType.DMA for _ in range(2)],
        [pltpu.SemaphoreType.DMA for _ in range(2)],
    ],
)
def _propagate(hs_hbm, src_hbm, dst_hbm, out_hbm,
               srci, dsti, rows, zbuf, acc, isem, gsem, ssem):
    c = lax.axis_index("c")
    s = lax.axis_index("s")
    wid = c * NS + s

    zero16 = jnp.zeros((16,), jnp.float32)

    def zbody(t, carry):
        zbuf[t // 8, pl.ds((t % 8) * 16, 16)] = zero16
        return carry

    lax.fori_loop(0, ZROWS * 8, zbody, 0)
    for j in range(RPT // ZROWS):
        pltpu.async_copy(zbuf, acc.at[pl.ds(s * RPT + j * ZROWS, ZROWS)],
                         gsem[0])
    for j in range(RPT // ZROWS):
        pltpu.make_async_copy(zbuf, acc.at[pl.ds(j * ZROWS, ZROWS)],
                              gsem[0]).wait()

    plsc.subcore_barrier()

    def ebase(t):
        return pl.multiple_of(wid * EPT + t * K, 8)

    def issue_idx(t, p):
        pltpu.async_copy(src_hbm.at[pl.ds(ebase(t), K)], srci[p], isem[p])
        pltpu.async_copy(dst_hbm.at[pl.ds(ebase(t), K)], dsti[p], isem[p])

    def issue_gather(t, b, p):
        pltpu.make_async_copy(src_hbm.at[pl.ds(ebase(t), K)], srci[p],
                              isem[p]).wait()
        pltpu.make_async_copy(dst_hbm.at[pl.ds(ebase(t), K)], dsti[p],
                              isem[p]).wait()
        pltpu.async_copy(hs_hbm.at[srci[p]], rows[b], gsem[b])

    def chunk(t, b, p, first=False, last=False, do_idx=True):
        pltpu.make_async_copy(hs_hbm.at[srci[p]], rows[b], gsem[b]).wait()
        pltpu.async_copy(rows[b], acc.at[dsti[p]], ssem[b], add=True)
        if not first:
            pltpu.make_async_copy(rows[1 - b], acc.at[dsti[(p + 3) % 4]],
                                  ssem[1 - b]).wait()
        if not last:
            issue_gather(t + 1, 1 - b, (p + 1) % 4)
            if do_idx:
                issue_idx(t + 3, (p + 3) % 4)

    issue_idx(0, 0)
    issue_idx(1, 1)
    issue_idx(2, 2)
    issue_gather(0, 0, 0)
    chunk(0, 0, 0, first=True)

    def mbody(u, carry):
        t0 = 1 + 4 * u
        for j in range(4):
            chunk(t0 + j, (1 + j) % 2, (1 + j) % 4)
        return carry

    lax.fori_loop(0, (NCHUNK - 2) // 4, mbody, 0)
    for t in range(NCHUNK - 4, NCHUNK):
        chunk(t, t % 2, t % 4, last=(t == NCHUNK - 1),
              do_idx=(t + 3 < NCHUNK))
    pltpu.make_async_copy(rows[(NCHUNK - 1) % 2],
                          acc.at[dsti[(NCHUNK - 1) % 4]],
                          ssem[(NCHUNK - 1) % 2]).wait()

    plsc.subcore_barrier()

    pltpu.sync_copy(acc.at[pl.ds(s * RPT, RPT)],
                    out_hbm.at[c, pl.ds(s * RPT, RPT)])


@functools.partial(
    pl.kernel,
    out_type=tuple(jax.ShapeDtypeStruct((NPAD * 8,), jnp.float32)
                   for _ in range(4)),
    mesh=_sc_mesh,
    scratch_types=[
        [pltpu.VMEM((K,), jnp.int32) for _ in range(4)],
        [pltpu.VMEM((K,), jnp.int32) for _ in range(4)],
        pltpu.VMEM((K,), jnp.float32),
        pltpu.VMEM((RPT,), jnp.float32),
        pltpu.VMEM((RPT * 8,), jnp.float32),
        pltpu.VMEM_SHARED((NPAD,), jnp.float32),
        pltpu.VMEM_SHARED((NPAD,), jnp.float32),
        [pltpu.SemaphoreType.DMA for _ in range(4)],
        [pltpu.SemaphoreType.DMA for _ in range(4)],
        [pltpu.SemaphoreType.DMA for _ in range(4)],
    ],
)
def _degrees(src_hbm, dst_hbm, outd0_hbm, outd1_hbm, outs0_hbm, outs1_hbm,
             ibd, ibs, ones, rtmp, rep, accd, accs, isem, dsem, ssem):
    c = lax.axis_index("c")
    s = lax.axis_index("s")
    wid = c * NS + s
    iota16 = lax.iota(jnp.int32, 16)
    zero16 = jnp.zeros((16,), jnp.float32)
    ones16 = jnp.ones((16,), jnp.float32)

    def ob(t, carry):
        ones[pl.ds(t * 16, 16)] = ones16
        return carry

    lax.fori_loop(0, K // 16, ob, 0)

    def zb(t, carry):
        rtmp[pl.ds(t * 16, 16)] = zero16
        return carry

    lax.fori_loop(0, RPT // 16, zb, 0)
    pltpu.sync_copy(rtmp, accd.at[pl.ds(s * RPT, RPT)])
    pltpu.sync_copy(rtmp, accs.at[pl.ds(s * RPT, RPT)])
    plsc.subcore_barrier()

    def ebase(t):
        return pl.multiple_of(wid * EPT + t * K, 8)

    def issue_idx(t, p):
        pltpu.async_copy(dst_hbm.at[pl.ds(ebase(t), K)], ibd[p], isem[p])
        pltpu.async_copy(src_hbm.at[pl.ds(ebase(t), K)], ibs[p], isem[p])

    def chunk(t, p, do_wait=True, do_issue=True):
        pltpu.make_async_copy(dst_hbm.at[pl.ds(ebase(t), K)], ibd[p],
                              isem[p]).wait()
        pltpu.make_async_copy(src_hbm.at[pl.ds(ebase(t), K)], ibs[p],
                              isem[p]).wait()
        pltpu.async_copy(ones, accd.at[ibd[p]], dsem[p], add=True)
        pltpu.async_copy(ones, accs.at[ibs[p]], ssem[p], add=True)
        p2 = (p + 2) % 4
        if do_wait:
            pltpu.make_async_copy(ones, accd.at[ibd[p2]], dsem[p2]).wait()
            pltpu.make_async_copy(ones, accs.at[ibs[p2]], ssem[p2]).wait()
        if do_issue:
            issue_idx(t + 2, p2)

    issue_idx(0, 0)
    issue_idx(1, 1)
    chunk(0, 0, do_wait=False)
    chunk(1, 1, do_wait=False)

    def mbody(u, carry):
        t0 = 2 + 4 * u
        for j in range(4):
            chunk(t0 + j, (2 + j) % 4)
        return carry

    lax.fori_loop(0, (NCHUNK - 5) // 4, mbody, 0)
    chunk(NCHUNK - 3, 2)
    chunk(NCHUNK - 2, 3, do_issue=False)
    chunk(NCHUNK - 1, 0, do_issue=False)
    for p in (3, 0):
        pltpu.make_async_copy(ones, accd.at[ibd[p]], dsem[p]).wait()
        pltpu.make_async_copy(ones, accs.at[ibs[p]], ssem[p]).wait()

    plsc.subcore_barrier()

    def rw(acc, out0_hbm, out1_hbm):
        pltpu.sync_copy(acc.at[pl.ds(s * RPT, RPT)], rtmp)

        def rb(q, carry):
            v = rtmp[pl.ds(q * 16, 16)]
            for r in range(8):
                rv = jnp.where(iota16 < 8, v[2 * r], v[2 * r + 1])
                rep[pl.ds(q * 128 + r * 16, 16)] = rv
            return carry

        lax.fori_loop(0, RPT // 16, rb, 0)
        stripe = pl.ds(s * RPT * 8, RPT * 8)

        @pl.when(c == 0)
        def _w0():
            pltpu.sync_copy(rep, out0_hbm.at[stripe])

        @pl.when(c == 1)
        def _w1():
            pltpu.sync_copy(rep, out1_hbm.at[stripe])

    rw(accd, outd0_hbm, outd1_hbm)
    rw(accs, outs0_hbm, outs1_hbm)


def _vsum(v):
    t = v[0]
    for j in range(1, 16):
        t = t + v[j]
    return t


def _prefix(v, iota16):
    acc = v[0]
    out = jnp.where(iota16 == 0, acc, 0.0)
    for j in range(1, 16):
        acc = acc + v[j]
        out = out + jnp.where(iota16 == j, acc, 0.0)
    return out


CAP = 8192
HTOT = 8448
CHK = N // LAYERS
THR0 = float(N - CHK)
THR1 = float(N - 2 * CHK)


@functools.partial(
    pl.kernel,
    out_type=jax.ShapeDtypeStruct((NPAD * 8,), jnp.float32),
    mesh=_sc_mesh,
    scratch_types=[
        pltpu.VMEM((RPT * 8,), jnp.float32),
        pltpu.VMEM((RPT,), jnp.float32),
        pltpu.VMEM((CAP,), jnp.float32),
        pltpu.VMEM((CAP,), jnp.float32),
        pltpu.VMEM((RPT * 8,), jnp.float32),
        pltpu.VMEM((16,), jnp.float32),
        pltpu.VMEM((16,), jnp.float32),
        pltpu.VMEM((NS * 16,), jnp.float32),
        pltpu.VMEM((16,), jnp.int32),
        pltpu.VMEM_SHARED((HTOT,), jnp.float32),
        pltpu.VMEM_SHARED((16,), jnp.float32),
        pltpu.VMEM_SHARED((NS * 16,), jnp.float32),
    ],
)
def _centrality(deg_hbm, out_hbm,
                degbuf, degv, histbuf, sbuf, outbuf, tmp16, ones1,
                cntbuf, ivbuf, hist, bcast, counts):
    c = lax.axis_index("c")
    s = lax.axis_index("s")
    zero16 = jnp.zeros((16,), jnp.float32)
    iota16 = lax.iota(jnp.int32, 16)

    @pl.when(c == 0)
    def _stage():
        tmp16[...] = zero16

        def zh(t, carry):
            pltpu.sync_copy(tmp16, hist.at[pl.ds(s * (HTOT // NS) + t * 16,
                                                 16)])
            return carry

        lax.fori_loop(0, HTOT // NS // 16, zh, 0)
        pltpu.sync_copy(deg_hbm.at[pl.ds(s * RPT * 8, RPT * 8)], degbuf)
        ones1[...] = jnp.ones((16,), jnp.float32)

        def ex(q, carry):
            v = jnp.zeros((16,), jnp.float32)
            for j in range(8):
                w = degbuf[pl.ds(q * 128 + j * 16, 16)]
                v = v + jnp.where(iota16 == 2 * j, w[0], 0.0)
                v = v + jnp.where(iota16 == 2 * j + 1, w[8], 0.0)
            degv[pl.ds(q * 16, 16)] = v
            return carry

        lax.fori_loop(0, RPT // 16, ex, 0)

    plsc.subcore_barrier()

    @pl.when(c == 0)
    def _histo():
        def hb(q, carry):
            v = degv[pl.ds(q * 16, 16)]
            ivbuf[...] = jnp.minimum(v.astype(jnp.int32), CAP)
            pltpu.sync_copy(ones1, hist.at[ivbuf], add=True)
            return carry

        lax.fori_loop(0, RPT // 16, hb, 0)

    plsc.subcore_barrier()

    @pl.when(jnp.logical_and(c == 0, s == 0))
    def _scan():
        pltpu.sync_copy(hist.at[pl.ds(0, CAP)], histbuf)

        def sb(q, carry):
            sacc, t0a, t1a = carry
            hv = histbuf[pl.ds(q * 16, 16)]
            sl = _prefix(hv, iota16) + sacc
            sbuf[pl.ds(q * 16, 16)] = sl
            t0a = t0a + jnp.where(sl <= THR0, 1.0, 0.0)
            t1a = t1a + jnp.where(sl <= THR1, 1.0, 0.0)
            return (sl[15], t0a, t1a)

        sacc, t0a, t1a = lax.fori_loop(0, CAP // 16, sb,
                                       (0.0, zero16, zero16))
        t0 = _vsum(t0a)
        t1 = _vsum(t1a)
        def _sread(tf):
            ti = tf.astype(jnp.int32)
            base = (ti // 16) * 16
            lane = ti - base
            v = sbuf[pl.ds(base, 16)]
            val = 0.0
            for j in range(16):
                val = jnp.where(lane == j, v[j], val)
            return val

        cgt0 = float(N) - _sread(t0)
        cgt1 = float(N) - _sread(t1)
        vec = jnp.where(iota16 == 0, t0,
                        jnp.where(iota16 == 1, t1,
                                  jnp.where(iota16 == 2, cgt0, cgt1)))
        tmp16[...] = vec
        pltpu.sync_copy(tmp16, bcast)

    plsc.subcore_barrier()

    @pl.when(c == 0)
    def _counts():
        pltpu.sync_copy(bcast, tmp16)
        tv = tmp16[...]
        t0f = tv[0]
        t1f = tv[1]

        def cb(q, carry):
            e0a, e1a = carry
            v = degv[pl.ds(q * 16, 16)]
            e0a = e0a + jnp.where(v == t0f, 1.0, 0.0)
            e1a = e1a + jnp.where(v == t1f, 1.0, 0.0)
            return (e0a, e1a)

        e0a, e1a = lax.fori_loop(0, RPT // 16, cb, (zero16, zero16))
        e0 = _vsum(e0a)
        e1 = _vsum(e1a)
        vec = jnp.where(iota16 == 0, e0, jnp.where(iota16 == 1, e1, 0.0))
        tmp16[...] = vec
        pltpu.sync_copy(tmp16, counts.at[pl.ds(s * 16, 16)])

    plsc.subcore_barrier()

    @pl.when(c == 0)
    def _assign():
        pltpu.sync_copy(counts, cntbuf)
        pltpu.sync_copy(bcast, tmp16)
        tv = tmp16[...]
        t0f = tv[0]
        t1f = tv[1]
        cgt0 = tv[2]
        cgt1 = tv[3]
        c0 = 0.0
        c1 = 0.0
        for t in range(NS):
            inrange = jnp.where(t < s, 1.0, 0.0)
            cv = cntbuf[pl.ds(t * 16, 16)]
            c0 = c0 + inrange * cv[0]
            c1 = c1 + inrange * cv[1]

        def fb(q, carry):
            c0, c1 = carry
            v = degv[pl.ds(q * 16, 16)]
            e0 = jnp.where(v == t0f, 1.0, 0.0)
            e1 = jnp.where(v == t1f, 1.0, 0.0)
            i0 = _prefix(e0, iota16)
            i1 = _prefix(e1, iota16)
            pos0 = c0 + i0 - e0
            pos1 = c1 + i1 - e1
            rank = jnp.where(e0 == 1.0, cgt0 + pos0,
                             jnp.where(e1 == 1.0, cgt1 + pos1,
                                       jnp.where(v > t0f, 0.0,
                                                 jnp.where(v < t1f,
                                                           9999.0, 4000.0))))
            ch = (jnp.where(rank >= float(CHK), 1.0, 0.0)
                  + jnp.where(rank >= float(2 * CHK), 1.0, 0.0))
            for r in range(8):
                outbuf[pl.ds(q * 128 + r * 16, 16)] = jnp.where(
                    iota16 < 8, ch[2 * r], ch[2 * r + 1])
            return (c0 + i0[15], c1 + i1[15])

        lax.fori_loop(0, RPT // 16, fb, (c0, c1))
        pltpu.sync_copy(outbuf, out_hbm.at[pl.ds(s * RPT * 8, RPT * 8)])


def _row_block(i):
    return (i, 0)


def _const_block(i):
    return (0, 0)


def _prelude_body(x_ref, wi_ref, bi_ref, w0_ref, dd0_ref, dd1_ref,
                  ds0_ref, ds1_ref,
                  h0_ref, hs0_ref, dinv_ref, degout_ref):
    i = pl.program_id(0)
    h0 = jnp.dot(x_ref[...], wi_ref[...],
                 preferred_element_type=jnp.float32) + bi_ref[...]
    h0_ref[...] = h0
    deg = dd0_ref[...] + dd1_ref[...]
    dinv1 = lax.rsqrt(deg[:, :1] + 1.0)
    dinv_ref[...] = jnp.broadcast_to(dinv1, (BLK, 8))
    hs0_ref[...] = jnp.dot(h0, w0_ref[...],
                           preferred_element_type=jnp.float32) * dinv1
    dsum = ds0_ref[...] + ds1_ref[...]
    row = i * BLK + lax.broadcasted_iota(jnp.int32, (BLK, 1), 0)
    degout_ref[...] = jnp.where(row < N, dsum, 1e9)


def _tc_prelude(x_pad, W_init, b_init, W0, dd0, dd1, ds0, ds1):
    return pl.pallas_call(
        _prelude_body,
        grid=(GRID,),
        in_specs=[
            pl.BlockSpec((BLK, D), _row_block),
            pl.BlockSpec((D, D), _const_block),
            pl.BlockSpec((1, D), _const_block),
            pl.BlockSpec((D, D), _const_block),
            pl.BlockSpec((BLK, 8), _row_block),
            pl.BlockSpec((BLK, 8), _row_block),
            pl.BlockSpec((BLK, 8), _row_block),
            pl.BlockSpec((BLK, 8), _row_block),
        ],
        out_specs=[
            pl.BlockSpec((BLK, D), _row_block),
            pl.BlockSpec((BLK, D), _row_block),
            pl.BlockSpec((BLK, 8), _row_block),
            pl.BlockSpec((BLK, 8), _row_block),
        ],
        out_shape=[
            jax.ShapeDtypeStruct((NPAD, D), jnp.float32),
            jax.ShapeDtypeStruct((NPAD, D), jnp.float32),
            jax.ShapeDtypeStruct((NPAD, 8), jnp.float32),
            jax.ShapeDtypeStruct((NPAD, 8), jnp.float32),
        ],
    )(x_pad, W_init, b_init, W0, dd0, dd1, ds0, ds1)


def _layer_body(parts_ref, hs_ref, h_ref, dinv_ref, chunk_ref,
                w_ref, b_ref, g_ref, be_ref, hn_ref, hsn_ref, *, layer):
    agg = parts_ref[0] + parts_ref[1] + hs_ref[...]
    dinv1 = dinv_ref[:, :1]
    cval = dinv1 * agg + b_ref[...]
    r = jnp.maximum(cval, 0.0)
    mu = jnp.mean(r, axis=1, keepdims=True)
    dev = r - mu
    var = jnp.mean(dev * dev, axis=1, keepdims=True)
    t = dev * lax.rsqrt(var + 1e-5) * g_ref[...] + be_ref[...]
    hn = jnp.where(chunk_ref[:, :1] == float(layer), t, h_ref[...])
    hn_ref[...] = hn
    hsn_ref[...] = jnp.dot(hn, w_ref[...],
                           preferred_element_type=jnp.float32) * dinv1


def _tc_layer(layer, parts, hs, h, dinv8, chunk8, W_next, b, g, be):
    return pl.pallas_call(
        functools.partial(_layer_body, layer=layer),
        grid=(GRID,),
        in_specs=[
            pl.BlockSpec((NC, BLK, D), lambda i: (0, i, 0)),
            pl.BlockSpec((BLK, D), _row_block),
            pl.BlockSpec((BLK, D), _row_block),
            pl.BlockSpec((BLK, 8), _row_block),
            pl.BlockSpec((BLK, 8), _row_block),
            pl.BlockSpec((D, D), _const_block),
            pl.BlockSpec((1, D), _const_block),
            pl.BlockSpec((1, D), _const_block),
            pl.BlockSpec((1, D), _const_block),
        ],
        out_specs=[
            pl.BlockSpec((BLK, D), _row_block),
            pl.BlockSpec((BLK, D), _row_block),
        ],
        out_shape=[
            jax.ShapeDtypeStruct((NPAD, D), jnp.float32),
            jax.ShapeDtypeStruct((NPAD, D), jnp.float32),
        ],
    )(parts, hs, h, dinv8, chunk8, W_next, b, g, be)


def _final_body(parts_ref, hs_ref, h_ref, dinv_ref, chunk_ref, b_ref,
                wc_ref, bc_ref, pool_ref, out_ref):
    i = pl.program_id(0)
    agg = parts_ref[0] + parts_ref[1] + hs_ref[...]
    cval = dinv_ref[:, :1] * agg + b_ref[...]
    hn = jnp.where(chunk_ref[:, :1] == 2.0, cval, h_ref[...])
    row = i * BLK + lax.broadcasted_iota(jnp.int32, (BLK, 1), 0)
    hm = jnp.where(row < N, hn, 0.0)
    part = jnp.sum(hm, axis=0, keepdims=True)

    @pl.when(i == 0)
    def _():
        pool_ref[...] = jnp.zeros_like(pool_ref)

    pool_ref[0:1, :] += part

    @pl.when(i == GRID - 1)
    def _():
        pooled = pool_ref[0:1, :] * (1.0 / N)
        logits = jnp.dot(pooled, wc_ref[...],
                         preferred_element_type=jnp.float32) + bc_ref[...]
        m = jnp.max(logits, axis=1, keepdims=True)
        z = logits - m
        lse = jnp.log(jnp.sum(jnp.exp(z), axis=1, keepdims=True)) + m
        out_ref[...] = logits - lse


def _tc_final(parts, hs, h, dinv8, chunk8, b, W_cls, b_cls):
    return pl.pallas_call(
        _final_body,
        grid=(GRID,),
        in_specs=[
            pl.BlockSpec((NC, BLK, D), lambda i: (0, i, 0)),
            pl.BlockSpec((BLK, D), _row_block),
            pl.BlockSpec((BLK, D), _row_block),
            pl.BlockSpec((BLK, 8), _row_block),
            pl.BlockSpec((BLK, 8), _row_block),
            pl.BlockSpec((1, D), _const_block),
            pl.BlockSpec((D, D), _const_block),
            pl.BlockSpec((1, D), _const_block),
        ],
        out_specs=[pl.BlockSpec((8, D), _const_block),
                   pl.BlockSpec((1, D), _const_block)],
        out_shape=[jax.ShapeDtypeStruct((8, D), jnp.float32),
                   jax.ShapeDtypeStruct((1, D), jnp.float32)],
    )(parts, hs, h, dinv8, chunk8, b, W_cls, b_cls)


def kernel(x, edge_index, batch, ptr, y, W_init, b_init, W0, b0, W1, b1,
           W2, b2, g0, be0, g1, be1, W_cls, b_cls):
    src = edge_index[0]
    dst = edge_index[1]
    x_pad = jnp.pad(x, ((0, NPAD - N), (0, 0)))

    dd0, dd1, ds0, ds1 = _degrees(src, dst)
    dd0 = dd0.reshape(NPAD, 8)
    dd1 = dd1.reshape(NPAD, 8)
    ds0 = ds0.reshape(NPAD, 8)
    ds1 = ds1.reshape(NPAD, 8)

    bi = b_init.reshape(1, D)
    h, hs, dinv8, degout8 = _tc_prelude(x_pad, W_init, bi, W0,
                                        dd0, dd1, ds0, ds1)
    chunk8 = _centrality(degout8.reshape(NPAD * 8)).reshape(NPAD, 8)

    lns = [(g0, be0), (g1, be1)]
    nxt = [W1, W2]
    bs = [b0, b1, b2]
    for i in range(2):
        parts = _propagate(hs, src, dst)
        h, hs = _tc_layer(i, parts, hs, h, dinv8, chunk8,
                          nxt[i], bs[i].reshape(1, D),
                          lns[i][0].reshape(1, D), lns[i][1].reshape(1, D))
    parts = _propagate(hs, src, dst)
    _pool, out = _tc_final(parts, hs, h, dinv8, chunk8, b2.reshape(1, D),
                           W_cls, b_cls.reshape(1, D))
    return (out, y)
